# Initial kernel scaffold; baseline (speedup 1.0000x reference)
import jax
import jax.numpy as jnp
from jax.experimental import pallas as pl


def kernel(x, params, current_epoch):
    raise NotImplementedError("write your pallas kernel here")


# speedup vs baseline: 1.7018x; 1.7018x over previous
import functools

import jax
import jax.numpy as jnp
from jax import lax
from jax.experimental import pallas as pl

DIM = 256
DI = 512
DS = 16
DR = 16
DC = 4
NH = 4

_PREC = lax.Precision.HIGHEST


def _dot(a, b):
    return jnp.dot(a, b, precision=_PREC, preferred_element_type=jnp.float32)


def _scores_kernel(x2d_ref, w1_ref, b1_ref, w2_ref, b2_ref,
                   scr_ref, scc_ref, *, n_tok, h, w, t_len, c_dim):
    x = x2d_ref[...]
    xm = jnp.mean(x.reshape(t_len, c_dim, n_tok), axis=0)

    col = lax.broadcasted_iota(jnp.int32, (1, n_tok), 1)
    xcol = jnp.mod(col, w)

    h1 = jnp.zeros((w1_ref.shape[0] // 9, n_tok), jnp.float32)
    o_idx = 0
    oc = w1_ref.shape[0] // 9
    for dy in (-1, 0, 1):
        for dx in (-1, 0, 1):
            s = dy * w + dx
            if s > 0:
                shifted = jnp.concatenate(
                    [xm[:, s:], jnp.zeros((c_dim, s), jnp.float32)], axis=1)
            elif s < 0:
                shifted = jnp.concatenate(
                    [jnp.zeros((c_dim, -s), jnp.float32), xm[:, :n_tok + s]],
                    axis=1)
            else:
                shifted = xm
            if dx == -1:
                shifted = jnp.where(xcol == 0, 0.0, shifted)
            elif dx == 1:
                shifted = jnp.where(xcol == w - 1, 0.0, shifted)
            wblk = w1_ref[o_idx * oc:(o_idx + 1) * oc, :]
            h1 = h1 + _dot(wblk, shifted)
            o_idx += 1
    h1 = h1 + b1_ref[...].reshape(-1, 1)
    h1 = jnp.where(h1 >= 0, h1, 0.01 * h1)
    w2 = w2_ref[...]
    s_lin = jnp.sum(h1 * w2.reshape(-1, 1), axis=0, keepdims=True)
    s_lin = s_lin + b2_ref[0, 0]
    scr_ref[...] = jax.nn.sigmoid(s_lin)
    s_col = lax.dot_general(h1, w2, (((0,), (0,)), ((), ())),
                            precision=_PREC,
                            preferred_element_type=jnp.float32)
    scc_ref[...] = jax.nn.sigmoid(s_col + b2_ref[0, 0])


def _mask_kernel(scr_ref, scc_ref, mrow_ref, mcol_ref, *, n_tok, k_sel, rb):
    pid = pl.program_id(0)
    sb = scr_ref[0:1, pl.ds(pid * rb, rb)]
    scc = scc_ref[...]
    jj = lax.broadcasted_iota(jnp.int32, (n_tok, rb), 0)
    ig = lax.broadcasted_iota(jnp.int32, (n_tok, rb), 1) + pid * rb
    beats = (scc > sb) | ((scc == sb) & (jj < ig))
    rank = jnp.sum(beats.astype(jnp.float32), axis=0, keepdims=True)
    m = (rank < k_sel).astype(jnp.float32)
    mrow_ref[...] = m
    mcol_ref[...] = jnp.transpose(m, (1, 0))


def _pos_kernel(mcol_ref, prow_ref, *, n_tok, rb):
    pid = pl.program_id(0)
    mc = mcol_ref[...]
    jj = lax.broadcasted_iota(jnp.int32, (n_tok, rb), 0)
    ig = lax.broadcasted_iota(jnp.int32, (n_tok, rb), 1) + pid * rb
    before = jnp.where(jj < ig, mc, 0.0)
    prow_ref[...] = jnp.sum(before, axis=0, keepdims=True)


def _oh_kernel(scr_ref, mrow_ref, prow_ref, oh_ref, ohs_ref, *, n_tok, kb):
    pid = pl.program_id(0)
    scr = scr_ref[...]
    mrow = mrow_ref[...]
    prow = prow_ref[...]
    kk = (lax.broadcasted_iota(jnp.int32, (kb, n_tok), 0)
          + pid * kb).astype(jnp.float32)
    oh = jnp.where((prow == kk) & (mrow > 0), 1.0, 0.0)
    sg = jnp.sum(oh * scr, axis=1, keepdims=True)
    oh_ref[...] = oh
    ohs_ref[...] = oh * (sg / (sg + 1e-6))


def _mamba_kernel(ohs_ref, xnm_ref, n1w_ref, inw_ref, cw_ref, cb_ref,
                  xpw_ref, dtw_ref, dtb_ref, alog_ref, d_ref, outw_ref,
                  z_ref, *, kb, t_len, c_dim):
    oh = ohs_ref[...]
    xg = _dot(oh, xnm_ref[...])
    u = xg.reshape(kb * t_len, c_dim)
    ms = jnp.mean(u * u, axis=1, keepdims=True)
    z = u * lax.rsqrt(ms + 1e-5) * n1w_ref[...].reshape(1, c_dim)
    xz = _dot(z, inw_ref[...])
    xc_pre = xz[:, :DI]
    zg = xz[:, DI:]
    xc3 = xc_pre.reshape(kb, t_len, DI)
    xp = jnp.concatenate(
        [jnp.zeros((kb, DC - 1, DI), jnp.float32), xc3], axis=1)
    cw = cw_ref[...]
    cb = cb_ref[...].reshape(1, 1, DI)
    acc = cb
    for j in range(DC):
        acc = acc + xp[:, j:j + t_len, :] * cw[:, j].reshape(1, 1, DI)
    xc3 = acc * jax.nn.sigmoid(acc)
    xc = xc3.reshape(kb * t_len, DI)
    x_dbl = _dot(xc, xpw_ref[...])
    dt_in = x_dbl[:, :DR]
    bs = x_dbl[:, DR:DR + DS].reshape(kb, t_len, DS)
    cs = x_dbl[:, DR + DS:DR + 2 * DS].reshape(kb, t_len, DS)
    dt = jax.nn.softplus(_dot(dt_in, dtw_ref[...]) + dtb_ref[...].reshape(1, DI))
    dt3 = dt.reshape(kb, t_len, DI)
    a_mat = -jnp.exp(alog_ref[...])
    xcb = xc.reshape(kb, t_len, DI)

    h = jnp.zeros((kb, DS, DI), jnp.float32)
    ys = []
    for t in range(t_len):
        dtt = dt3[:, t, :]
        da = jnp.exp(dtt[:, None, :] * a_mat[None, :, :])
        dbx = (dtt * xcb[:, t, :])[:, None, :] * bs[:, t, :, None]
        h = da * h + dbx
        ys.append(jnp.sum(h * cs[:, t, :, None], axis=1))
    y = jnp.stack(ys, axis=1).reshape(kb * t_len, DI)
    y = y + xc * d_ref[...].reshape(1, DI)
    y = y * (zg * jax.nn.sigmoid(zg))
    zout = _dot(y, outw_ref[...])
    z_ref[...] = zout.reshape(kb, t_len, c_dim).transpose(1, 0, 2)


def _attn_kernel(z_ref, aiw_ref, aib_ref, aow_ref, aob_ref,
                 lnw_ref, lnb_ref, oh_ref, x2d_ref, out_ref,
                 *, k_sel, c_dim, n_tok):
    zs = z_ref[...].reshape(k_sel, c_dim)
    qkv = _dot(zs, aiw_ref[...]) + aib_ref[...].reshape(1, -1)
    dh = c_dim // NH
    inv = 1.0 / jnp.sqrt(jnp.float32(dh))
    outs = []
    for hh in range(NH):
        q = qkv[:, hh * dh:(hh + 1) * dh]
        k = qkv[:, c_dim + hh * dh:c_dim + (hh + 1) * dh]
        v = qkv[:, 2 * c_dim + hh * dh:2 * c_dim + (hh + 1) * dh]
        att = lax.dot_general(q, k, (((1,), (1,)), ((), ())),
                              precision=_PREC,
                              preferred_element_type=jnp.float32) * inv
        att = att - jnp.max(att, axis=1, keepdims=True)
        att = jnp.exp(att)
        att = att / jnp.sum(att, axis=1, keepdims=True)
        outs.append(_dot(att, v))
    o = jnp.concatenate(outs, axis=1)
    o = _dot(o, aow_ref[...]) + aob_ref[...].reshape(1, c_dim) + zs
    mu = jnp.mean(o, axis=1, keepdims=True)
    var = jnp.mean((o - mu) ** 2, axis=1, keepdims=True)
    zo = (o - mu) * lax.rsqrt(var + 1e-5)
    zo = zo * lnw_ref[...].reshape(1, c_dim) + lnb_ref[...].reshape(1, c_dim)
    dense = lax.dot_general(zo, oh_ref[...], (((0,), (0,)), ((), ())),
                            precision=_PREC,
                            preferred_element_type=jnp.float32)
    out_ref[...] = (x2d_ref[...].reshape(c_dim, n_tok) + dense).reshape(
        1, c_dim, n_tok)


def kernel(x, params, current_epoch):
    b, t_len, c_dim, h, w = x.shape
    n_tok = h * w
    k_sel = max(1, n_tok // 2)
    p = params

    x2d = x.reshape(t_len * c_dim, n_tok)
    x_nm = x2d.T
    w1 = p['r_w1'].transpose(2, 3, 0, 1).reshape(9 * p['r_w1'].shape[0],
                                                 c_dim)

    scr, scc = pl.pallas_call(
        functools.partial(_scores_kernel, n_tok=n_tok, h=h, w=w,
                          t_len=t_len, c_dim=c_dim),
        out_shape=(jax.ShapeDtypeStruct((1, n_tok), jnp.float32),
                   jax.ShapeDtypeStruct((n_tok, 1), jnp.float32)),
    )(x2d, w1, p['r_b1'].reshape(-1, 1), p['r_w2'].reshape(-1, 1),
      p['r_b2'].reshape(1, 1))

    rb = 128
    mrow, mcol = pl.pallas_call(
        functools.partial(_mask_kernel, n_tok=n_tok, k_sel=k_sel, rb=rb),
        grid=(n_tok // rb,),
        in_specs=[
            pl.BlockSpec((1, n_tok), lambda i: (0, 0)),
            pl.BlockSpec((n_tok, 1), lambda i: (0, 0)),
        ],
        out_specs=(pl.BlockSpec((1, rb), lambda i: (0, i)),
                   pl.BlockSpec((rb, 1), lambda i: (i, 0))),
        out_shape=(jax.ShapeDtypeStruct((1, n_tok), jnp.float32),
                   jax.ShapeDtypeStruct((n_tok, 1), jnp.float32)),
    )(scr, scc)

    prow = pl.pallas_call(
        functools.partial(_pos_kernel, n_tok=n_tok, rb=rb),
        grid=(n_tok // rb,),
        in_specs=[pl.BlockSpec((n_tok, 1), lambda i: (0, 0))],
        out_specs=pl.BlockSpec((1, rb), lambda i: (0, i)),
        out_shape=jax.ShapeDtypeStruct((1, n_tok), jnp.float32),
    )(mcol)

    kb2 = 128
    oh, ohs = pl.pallas_call(
        functools.partial(_oh_kernel, n_tok=n_tok, kb=kb2),
        grid=(k_sel // kb2,),
        in_specs=[
            pl.BlockSpec((1, n_tok), lambda i: (0, 0)),
            pl.BlockSpec((1, n_tok), lambda i: (0, 0)),
            pl.BlockSpec((1, n_tok), lambda i: (0, 0)),
        ],
        out_specs=(pl.BlockSpec((kb2, n_tok), lambda i: (i, 0)),
                   pl.BlockSpec((kb2, n_tok), lambda i: (i, 0))),
        out_shape=(jax.ShapeDtypeStruct((k_sel, n_tok), jnp.float32),
                   jax.ShapeDtypeStruct((k_sel, n_tok), jnp.float32)),
    )(scr, mrow, prow)

    kb = 64
    nblk = k_sel // kb
    z = pl.pallas_call(
        functools.partial(_mamba_kernel, kb=kb, t_len=t_len, c_dim=c_dim),
        grid=(nblk,),
        in_specs=[
            pl.BlockSpec((kb, n_tok), lambda i: (i, 0)),
            pl.BlockSpec((n_tok, t_len * c_dim), lambda i: (0, 0)),
            pl.BlockSpec((1, c_dim), lambda i: (0, 0)),
            pl.BlockSpec((c_dim, 2 * DI), lambda i: (0, 0)),
            pl.BlockSpec((DI, DC), lambda i: (0, 0)),
            pl.BlockSpec((1, DI), lambda i: (0, 0)),
            pl.BlockSpec((DI, DR + 2 * DS), lambda i: (0, 0)),
            pl.BlockSpec((DR, DI), lambda i: (0, 0)),
            pl.BlockSpec((1, DI), lambda i: (0, 0)),
            pl.BlockSpec((DS, DI), lambda i: (0, 0)),
            pl.BlockSpec((1, DI), lambda i: (0, 0)),
            pl.BlockSpec((DI, c_dim), lambda i: (0, 0)),
        ],
        out_specs=pl.BlockSpec((t_len, kb, c_dim), lambda i: (0, i, 0)),
        out_shape=jax.ShapeDtypeStruct((t_len, k_sel, c_dim), jnp.float32),
    )(ohs, x_nm, p['norm1_w'].reshape(1, c_dim), p['m_in_w'].T,
      p['m_conv_w'].reshape(DI, DC), p['m_conv_b'].reshape(1, DI),
      p['m_xproj_w'].T, p['m_dt_w'].T, p['m_dt_b'].reshape(1, DI),
      p['m_Alog'].T, p['m_D'].reshape(1, DI), p['m_out_w'].T)

    x3 = x2d.reshape(t_len, c_dim, n_tok)

    out = pl.pallas_call(
        functools.partial(_attn_kernel, k_sel=k_sel, c_dim=c_dim,
                          n_tok=n_tok),
        grid=(t_len,),
        in_specs=[
            pl.BlockSpec((1, k_sel, c_dim), lambda i: (i, 0, 0)),
            pl.BlockSpec((c_dim, 3 * c_dim), lambda i: (0, 0)),
            pl.BlockSpec((1, 3 * c_dim), lambda i: (0, 0)),
            pl.BlockSpec((c_dim, c_dim), lambda i: (0, 0)),
            pl.BlockSpec((1, c_dim), lambda i: (0, 0)),
            pl.BlockSpec((1, c_dim), lambda i: (0, 0)),
            pl.BlockSpec((1, c_dim), lambda i: (0, 0)),
            pl.BlockSpec((k_sel, n_tok), lambda i: (0, 0)),
            pl.BlockSpec((1, c_dim, n_tok), lambda i: (i, 0, 0)),
        ],
        out_specs=pl.BlockSpec((1, c_dim, n_tok), lambda i: (i, 0, 0)),
        out_shape=jax.ShapeDtypeStruct((t_len, c_dim, n_tok), jnp.float32),
    )(z, p['a_in_w'].T, p['a_in_b'].reshape(1, -1), p['a_out_w'].T,
      p['a_out_b'].reshape(1, c_dim), p['ln_w'].reshape(1, c_dim),
      p['ln_b'].reshape(1, c_dim), oh, x3)

    return out.reshape(b, t_len, c_dim, h, w)


# speedup vs baseline: 2.2460x; 1.3198x over previous
import functools

import jax
import jax.numpy as jnp
from jax import lax
from jax.experimental import pallas as pl

DIM = 256
DI = 512
DS = 16
DR = 16
DC = 4
NH = 4

_PREC = lax.Precision.HIGHEST
_PREC_MAIN = lax.Precision.DEFAULT


def _dot(a, b):
    return jnp.dot(a, b, precision=_PREC,
                   preferred_element_type=jnp.float32)


def _dotd(a, b):
    return jnp.dot(a, b, precision=_PREC_MAIN,
                   preferred_element_type=jnp.float32)


def _scores_kernel(x2d_ref, w1_ref, b1_ref, w2_ref, b2_ref,
                   scr_ref, scc_ref, *, n_tok, h, w, t_len, c_dim):
    x = x2d_ref[...]
    xm = jnp.mean(x.reshape(t_len, c_dim, n_tok), axis=0)

    col = lax.broadcasted_iota(jnp.int32, (1, n_tok), 1)
    xcol = jnp.mod(col, w)

    h1 = jnp.zeros((w1_ref.shape[0] // 9, n_tok), jnp.float32)
    o_idx = 0
    oc = w1_ref.shape[0] // 9
    for dy in (-1, 0, 1):
        for dx in (-1, 0, 1):
            s = dy * w + dx
            if s > 0:
                shifted = jnp.concatenate(
                    [xm[:, s:], jnp.zeros((c_dim, s), jnp.float32)], axis=1)
            elif s < 0:
                shifted = jnp.concatenate(
                    [jnp.zeros((c_dim, -s), jnp.float32), xm[:, :n_tok + s]],
                    axis=1)
            else:
                shifted = xm
            if dx == -1:
                shifted = jnp.where(xcol == 0, 0.0, shifted)
            elif dx == 1:
                shifted = jnp.where(xcol == w - 1, 0.0, shifted)
            wblk = w1_ref[o_idx * oc:(o_idx + 1) * oc, :]
            h1 = h1 + jnp.dot(wblk, shifted, precision=_PREC,
                              preferred_element_type=jnp.float32)
            o_idx += 1
    h1 = h1 + b1_ref[...].reshape(-1, 1)
    h1 = jnp.where(h1 >= 0, h1, 0.01 * h1)
    w2 = w2_ref[...]
    s_lin = jnp.sum(h1 * w2.reshape(-1, 1), axis=0, keepdims=True)
    s_lin = s_lin + b2_ref[0, 0]
    sc = jax.nn.sigmoid(s_lin)
    scr_ref[...] = sc
    scc_ref[...] = jnp.transpose(sc, (1, 0))


def _mask_kernel(scr_ref, scc_ref, mrow_ref, mcol_ref, *, n_tok, k_sel, rb):
    pid = pl.program_id(0)
    sb = scr_ref[0:1, pl.ds(pid * rb, rb)]
    scc = scc_ref[...]
    jj = lax.broadcasted_iota(jnp.int32, (n_tok, rb), 0)
    ig = lax.broadcasted_iota(jnp.int32, (n_tok, rb), 1) + pid * rb
    beats = (scc > sb) | ((scc == sb) & (jj < ig))
    rank = jnp.sum(beats.astype(jnp.float32), axis=0, keepdims=True)
    m = (rank < k_sel).astype(jnp.float32)
    mrow_ref[...] = m
    mcol_ref[...] = jnp.transpose(m, (1, 0))


def _pos_kernel(mcol_ref, prow_ref, *, n_tok, rb):
    pid = pl.program_id(0)
    mc = mcol_ref[...]
    jj = lax.broadcasted_iota(jnp.int32, (n_tok, rb), 0)
    ig = lax.broadcasted_iota(jnp.int32, (n_tok, rb), 1) + pid * rb
    before = jnp.where(jj < ig, mc, 0.0)
    prow_ref[...] = jnp.sum(before, axis=0, keepdims=True)


def _oh_kernel(scr_ref, mrow_ref, prow_ref, oh_ref, ohs_ref, *, n_tok, kb):
    pid = pl.program_id(0)
    scr = scr_ref[...]
    mrow = mrow_ref[...]
    prow = prow_ref[...]
    kk = (lax.broadcasted_iota(jnp.int32, (kb, n_tok), 0)
          + pid * kb).astype(jnp.float32)
    oh = jnp.where((prow == kk) & (mrow > 0), 1.0, 0.0)
    sg = jnp.sum(oh * scr, axis=1, keepdims=True)
    oh_ref[...] = oh
    ohs_ref[...] = oh * (sg / (sg + 1e-6))


def _mamba_kernel(ohs_ref, xnm_ref, n1w_ref, inw_ref, cw_ref, cb_ref,
                  xpw_ref, dtw_ref, dtb_ref, alog_ref, d_ref, outw_ref,
                  z_ref, *, kb, t_len, c_dim):
    oh = ohs_ref[...]
    xg = _dot(oh, xnm_ref[...])
    u = xg.reshape(kb * t_len, c_dim)
    ms = jnp.mean(u * u, axis=1, keepdims=True)
    z = u * lax.rsqrt(ms + 1e-5) * n1w_ref[...].reshape(1, c_dim)
    xz = _dot(z, inw_ref[...])
    xc_pre = xz[:, :DI]
    zg = xz[:, DI:]
    xc3 = xc_pre.reshape(kb, t_len, DI)
    xp = jnp.concatenate(
        [jnp.zeros((kb, DC - 1, DI), jnp.float32), xc3], axis=1)
    cw = cw_ref[...]
    cb = cb_ref[...].reshape(1, 1, DI)
    acc = cb
    for j in range(DC):
        acc = acc + xp[:, j:j + t_len, :] * cw[:, j].reshape(1, 1, DI)
    xc3 = acc * jax.nn.sigmoid(acc)
    xc = xc3.reshape(kb * t_len, DI)
    x_dbl = _dot(xc, xpw_ref[...])
    dt_in = x_dbl[:, :DR]
    bs = x_dbl[:, DR:DR + DS].reshape(kb, t_len, DS)
    cs = x_dbl[:, DR + DS:DR + 2 * DS].reshape(kb, t_len, DS)
    dt = jax.nn.softplus(_dot(dt_in, dtw_ref[...]) + dtb_ref[...].reshape(1, DI))
    dt3 = dt.reshape(kb, t_len, DI)
    a_mat = -jnp.exp(alog_ref[...])
    xcb = xc.reshape(kb, t_len, DI)

    h = jnp.zeros((kb, DS, DI), jnp.float32)
    ys = []
    for t in range(t_len):
        dtt = dt3[:, t, :]
        da = jnp.exp(dtt[:, None, :] * a_mat[None, :, :])
        dbx = (dtt * xcb[:, t, :])[:, None, :] * bs[:, t, :, None]
        h = da * h + dbx
        ys.append(jnp.sum(h * cs[:, t, :, None], axis=1))
    y = jnp.stack(ys, axis=1).reshape(kb * t_len, DI)
    y = y + xc * d_ref[...].reshape(1, DI)
    y = y * (zg * jax.nn.sigmoid(zg))
    zout = _dotd(y, outw_ref[...])
    z_ref[...] = zout.reshape(kb, t_len, c_dim).transpose(1, 0, 2)


def _attn_kernel(z_ref, aiw_ref, aib_ref, aow_ref, aob_ref,
                 lnw_ref, lnb_ref, oh_ref, x2d_ref, out_ref,
                 *, k_sel, c_dim, n_tok):
    zs = z_ref[...].reshape(k_sel, c_dim)
    qkv = _dotd(zs, aiw_ref[...]) + aib_ref[...].reshape(1, -1)
    dh = c_dim // NH
    inv = 1.0 / jnp.sqrt(jnp.float32(dh))
    outs = []
    for hh in range(NH):
        q = qkv[:, hh * dh:(hh + 1) * dh]
        k = qkv[:, c_dim + hh * dh:c_dim + (hh + 1) * dh]
        v = qkv[:, 2 * c_dim + hh * dh:2 * c_dim + (hh + 1) * dh]
        att = lax.dot_general(q, k, (((1,), (1,)), ((), ())),
                              precision=_PREC_MAIN,
                              preferred_element_type=jnp.float32) * inv
        att = att - jnp.max(att, axis=1, keepdims=True)
        att = jnp.exp(att)
        att = att / jnp.sum(att, axis=1, keepdims=True)
        outs.append(_dotd(att, v))
    o = jnp.concatenate(outs, axis=1)
    o = _dotd(o, aow_ref[...]) + aob_ref[...].reshape(1, c_dim) + zs
    mu = jnp.mean(o, axis=1, keepdims=True)
    var = jnp.mean((o - mu) ** 2, axis=1, keepdims=True)
    zo = (o - mu) * lax.rsqrt(var + 1e-5)
    zo = zo * lnw_ref[...].reshape(1, c_dim) + lnb_ref[...].reshape(1, c_dim)
    dense = lax.dot_general(zo, oh_ref[...], (((0,), (0,)), ((), ())),
                            precision=_PREC_MAIN,
                            preferred_element_type=jnp.float32)
    out_ref[...] = (x2d_ref[...].reshape(c_dim, n_tok) + dense).reshape(
        1, c_dim, n_tok)


def kernel(x, params, current_epoch):
    b, t_len, c_dim, h, w = x.shape
    n_tok = h * w
    k_sel = max(1, n_tok // 2)
    p = params

    x2d = x.reshape(t_len * c_dim, n_tok)
    x_nm = x2d.T
    w1 = p['r_w1'].transpose(2, 3, 0, 1).reshape(9 * p['r_w1'].shape[0],
                                                 c_dim)

    scr, scc = pl.pallas_call(
        functools.partial(_scores_kernel, n_tok=n_tok, h=h, w=w,
                          t_len=t_len, c_dim=c_dim),
        out_shape=(jax.ShapeDtypeStruct((1, n_tok), jnp.float32),
                   jax.ShapeDtypeStruct((n_tok, 1), jnp.float32)),
    )(x2d, w1, p['r_b1'].reshape(-1, 1), p['r_w2'].reshape(-1, 1),
      p['r_b2'].reshape(1, 1))

    rb = 128
    mrow, mcol = pl.pallas_call(
        functools.partial(_mask_kernel, n_tok=n_tok, k_sel=k_sel, rb=rb),
        grid=(n_tok // rb,),
        in_specs=[
            pl.BlockSpec((1, n_tok), lambda i: (0, 0)),
            pl.BlockSpec((n_tok, 1), lambda i: (0, 0)),
        ],
        out_specs=(pl.BlockSpec((1, rb), lambda i: (0, i)),
                   pl.BlockSpec((rb, 1), lambda i: (i, 0))),
        out_shape=(jax.ShapeDtypeStruct((1, n_tok), jnp.float32),
                   jax.ShapeDtypeStruct((n_tok, 1), jnp.float32)),
    )(scr, scc)

    prow = pl.pallas_call(
        functools.partial(_pos_kernel, n_tok=n_tok, rb=rb),
        grid=(n_tok // rb,),
        in_specs=[pl.BlockSpec((n_tok, 1), lambda i: (0, 0))],
        out_specs=pl.BlockSpec((1, rb), lambda i: (0, i)),
        out_shape=jax.ShapeDtypeStruct((1, n_tok), jnp.float32),
    )(mcol)

    kb2 = 128
    oh, ohs = pl.pallas_call(
        functools.partial(_oh_kernel, n_tok=n_tok, kb=kb2),
        grid=(k_sel // kb2,),
        in_specs=[
            pl.BlockSpec((1, n_tok), lambda i: (0, 0)),
            pl.BlockSpec((1, n_tok), lambda i: (0, 0)),
            pl.BlockSpec((1, n_tok), lambda i: (0, 0)),
        ],
        out_specs=(pl.BlockSpec((kb2, n_tok), lambda i: (i, 0)),
                   pl.BlockSpec((kb2, n_tok), lambda i: (i, 0))),
        out_shape=(jax.ShapeDtypeStruct((k_sel, n_tok), jnp.float32),
                   jax.ShapeDtypeStruct((k_sel, n_tok), jnp.float32)),
    )(scr, mrow, prow)

    kb = 64
    nblk = k_sel // kb
    z = pl.pallas_call(
        functools.partial(_mamba_kernel, kb=kb, t_len=t_len, c_dim=c_dim),
        grid=(nblk,),
        in_specs=[
            pl.BlockSpec((kb, n_tok), lambda i: (i, 0)),
            pl.BlockSpec((n_tok, t_len * c_dim), lambda i: (0, 0)),
            pl.BlockSpec((1, c_dim), lambda i: (0, 0)),
            pl.BlockSpec((c_dim, 2 * DI), lambda i: (0, 0)),
            pl.BlockSpec((DI, DC), lambda i: (0, 0)),
            pl.BlockSpec((1, DI), lambda i: (0, 0)),
            pl.BlockSpec((DI, DR + 2 * DS), lambda i: (0, 0)),
            pl.BlockSpec((DR, DI), lambda i: (0, 0)),
            pl.BlockSpec((1, DI), lambda i: (0, 0)),
            pl.BlockSpec((DS, DI), lambda i: (0, 0)),
            pl.BlockSpec((1, DI), lambda i: (0, 0)),
            pl.BlockSpec((DI, c_dim), lambda i: (0, 0)),
        ],
        out_specs=pl.BlockSpec((t_len, kb, c_dim), lambda i: (0, i, 0)),
        out_shape=jax.ShapeDtypeStruct((t_len, k_sel, c_dim), jnp.float32),
    )(ohs, x_nm, p['norm1_w'].reshape(1, c_dim), p['m_in_w'].T,
      p['m_conv_w'].reshape(DI, DC), p['m_conv_b'].reshape(1, DI),
      p['m_xproj_w'].T, p['m_dt_w'].T, p['m_dt_b'].reshape(1, DI),
      p['m_Alog'].T, p['m_D'].reshape(1, DI), p['m_out_w'].T)

    x3 = x2d.reshape(t_len, c_dim, n_tok)

    out = pl.pallas_call(
        functools.partial(_attn_kernel, k_sel=k_sel, c_dim=c_dim,
                          n_tok=n_tok),
        grid=(t_len,),
        in_specs=[
            pl.BlockSpec((1, k_sel, c_dim), lambda i: (i, 0, 0)),
            pl.BlockSpec((c_dim, 3 * c_dim), lambda i: (0, 0)),
            pl.BlockSpec((1, 3 * c_dim), lambda i: (0, 0)),
            pl.BlockSpec((c_dim, c_dim), lambda i: (0, 0)),
            pl.BlockSpec((1, c_dim), lambda i: (0, 0)),
            pl.BlockSpec((1, c_dim), lambda i: (0, 0)),
            pl.BlockSpec((1, c_dim), lambda i: (0, 0)),
            pl.BlockSpec((k_sel, n_tok), lambda i: (0, 0)),
            pl.BlockSpec((1, c_dim, n_tok), lambda i: (i, 0, 0)),
        ],
        out_specs=pl.BlockSpec((1, c_dim, n_tok), lambda i: (i, 0, 0)),
        out_shape=jax.ShapeDtypeStruct((t_len, c_dim, n_tok), jnp.float32),
    )(z, p['a_in_w'].T, p['a_in_b'].reshape(1, -1), p['a_out_w'].T,
      p['a_out_b'].reshape(1, c_dim), p['ln_w'].reshape(1, c_dim),
      p['ln_b'].reshape(1, c_dim), oh, x3)

    return out.reshape(b, t_len, c_dim, h, w)


# speedup vs baseline: 2.4591x; 1.0949x over previous
import functools

import jax
import jax.numpy as jnp
from jax import lax
from jax.experimental import pallas as pl

DIM = 256
DI = 512
DS = 16
DR = 16
DC = 4
NH = 4

_PREC = lax.Precision.HIGHEST
_PREC_MAIN = lax.Precision.DEFAULT


def _dot(a, b):
    return jnp.dot(a, b, precision=_PREC,
                   preferred_element_type=jnp.float32)


def _dotd(a, b):
    return jnp.dot(a, b, precision=_PREC_MAIN,
                   preferred_element_type=jnp.float32)


def _scores_kernel(x2d_ref, w1_ref, b1_ref, w2_ref, b2_ref,
                   scr_ref, scc_ref, *, n_tok, h, w, t_len, c_dim):
    x = x2d_ref[...]
    xm = jnp.mean(x.reshape(t_len, c_dim, n_tok), axis=0)

    col = lax.broadcasted_iota(jnp.int32, (1, n_tok), 1)
    xcol = jnp.mod(col, w)

    h1 = jnp.zeros((w1_ref.shape[0] // 9, n_tok), jnp.float32)
    o_idx = 0
    oc = w1_ref.shape[0] // 9
    for dy in (-1, 0, 1):
        for dx in (-1, 0, 1):
            s = dy * w + dx
            if s > 0:
                shifted = jnp.concatenate(
                    [xm[:, s:], jnp.zeros((c_dim, s), jnp.float32)], axis=1)
            elif s < 0:
                shifted = jnp.concatenate(
                    [jnp.zeros((c_dim, -s), jnp.float32), xm[:, :n_tok + s]],
                    axis=1)
            else:
                shifted = xm
            if dx == -1:
                shifted = jnp.where(xcol == 0, 0.0, shifted)
            elif dx == 1:
                shifted = jnp.where(xcol == w - 1, 0.0, shifted)
            wblk = w1_ref[o_idx * oc:(o_idx + 1) * oc, :]
            h1 = h1 + jnp.dot(wblk, shifted, precision=_PREC,
                              preferred_element_type=jnp.float32)
            o_idx += 1
    h1 = h1 + b1_ref[...].reshape(-1, 1)
    h1 = jnp.where(h1 >= 0, h1, 0.01 * h1)
    w2 = w2_ref[...]
    s_lin = jnp.sum(h1 * w2.reshape(-1, 1), axis=0, keepdims=True)
    s_lin = s_lin + b2_ref[0, 0]
    sc = jax.nn.sigmoid(s_lin)
    scr_ref[...] = sc
    scc_ref[...] = jnp.transpose(sc, (1, 0))


def _mask_kernel(scr_ref, scc_ref, mrow_ref, mcol_ref, *, n_tok, k_sel, rb):
    pid = pl.program_id(0)
    sb = scr_ref[0:1, pl.ds(pid * rb, rb)]
    scc = scc_ref[...]
    jj = lax.broadcasted_iota(jnp.int32, (n_tok, rb), 0)
    ig = lax.broadcasted_iota(jnp.int32, (n_tok, rb), 1) + pid * rb
    beats = (scc > sb) | ((scc == sb) & (jj < ig))
    rank = jnp.sum(beats.astype(jnp.float32), axis=0, keepdims=True)
    m = (rank < k_sel).astype(jnp.float32)
    mrow_ref[...] = m
    mcol_ref[...] = jnp.transpose(m, (1, 0))


def _pos_kernel(mcol_ref, prow_ref, *, n_tok, rb):
    pid = pl.program_id(0)
    mc = mcol_ref[...]
    jj = lax.broadcasted_iota(jnp.int32, (n_tok, rb), 0)
    ig = lax.broadcasted_iota(jnp.int32, (n_tok, rb), 1) + pid * rb
    before = jnp.where(jj < ig, mc, 0.0)
    prow_ref[...] = jnp.sum(before, axis=0, keepdims=True)


def _oh_kernel(scr_ref, mrow_ref, prow_ref, oh_ref, ohs_ref, *, n_tok, kb):
    pid = pl.program_id(0)
    scr = scr_ref[...]
    mrow = mrow_ref[...]
    prow = prow_ref[...]
    kk = (lax.broadcasted_iota(jnp.int32, (kb, n_tok), 0)
          + pid * kb).astype(jnp.float32)
    oh = jnp.where((prow == kk) & (mrow > 0), 1.0, 0.0)
    sg = jnp.sum(oh * scr, axis=1, keepdims=True)
    oh_ref[...] = oh
    ohs_ref[...] = oh * (sg / (sg + 1e-6))


def _mamba_kernel(ohs_ref, xnm_ref, n1w_ref, inw_ref, cw_ref, cb_ref,
                  xpw_ref, dtw_ref, dtb_ref, alog_ref, d_ref, outw_ref,
                  z_ref, *, kb, t_len, c_dim):
    oh = ohs_ref[...]
    xnm = xnm_ref[...]
    xhi = xnm.astype(jnp.bfloat16)
    xlo = (xnm - xhi.astype(jnp.float32)).astype(jnp.bfloat16)
    ohb = oh.astype(jnp.bfloat16)
    xg = (jnp.dot(ohb, xhi, preferred_element_type=jnp.float32)
          + jnp.dot(ohb, xlo, preferred_element_type=jnp.float32))
    u = xg.reshape(kb * t_len, c_dim)
    ms = jnp.mean(u * u, axis=1, keepdims=True)
    z = u * lax.rsqrt(ms + 1e-5) * n1w_ref[...].reshape(1, c_dim)
    xz = _dot(z, inw_ref[...])
    xc_pre = xz[:, :DI]
    zg = xz[:, DI:]
    xc3 = xc_pre.reshape(kb, t_len, DI)
    xp = jnp.concatenate(
        [jnp.zeros((kb, DC - 1, DI), jnp.float32), xc3], axis=1)
    cw = cw_ref[...]
    cb = cb_ref[...].reshape(1, 1, DI)
    acc = cb
    for j in range(DC):
        acc = acc + xp[:, j:j + t_len, :] * cw[:, j].reshape(1, 1, DI)
    xc3 = acc * jax.nn.sigmoid(acc)
    xc = xc3.reshape(kb * t_len, DI)
    x_dbl = _dot(xc, xpw_ref[...])
    dt_in = x_dbl[:, :DR]
    bs = x_dbl[:, DR:DR + DS].reshape(kb, t_len, DS)
    cs = x_dbl[:, DR + DS:DR + 2 * DS].reshape(kb, t_len, DS)
    dt = jax.nn.softplus(_dot(dt_in, dtw_ref[...]) + dtb_ref[...].reshape(1, DI))
    dt3 = dt.reshape(kb, t_len, DI)
    a_mat = -jnp.exp(alog_ref[...])
    xcb = xc.reshape(kb, t_len, DI)

    h = jnp.zeros((kb, DS, DI), jnp.float32)
    ys = []
    for t in range(t_len):
        dtt = dt3[:, t, :]
        da = jnp.exp(dtt[:, None, :] * a_mat[None, :, :])
        dbx = (dtt * xcb[:, t, :])[:, None, :] * bs[:, t, :, None]
        h = da * h + dbx
        ys.append(jnp.sum(h * cs[:, t, :, None], axis=1))
    y = jnp.stack(ys, axis=1).reshape(kb * t_len, DI)
    y = y + xc * d_ref[...].reshape(1, DI)
    y = y * (zg * jax.nn.sigmoid(zg))
    zout = _dotd(y, outw_ref[...])
    z_ref[...] = zout.reshape(kb, t_len, c_dim).transpose(1, 0, 2)


def _attn_kernel(z_ref, aiw_ref, aib_ref, aow_ref, aob_ref,
                 lnw_ref, lnb_ref, oh_ref, x2d_ref, out_ref,
                 *, k_sel, c_dim, n_tok):
    zs = z_ref[...].reshape(k_sel, c_dim)
    qkv = _dotd(zs, aiw_ref[...]) + aib_ref[...].reshape(1, -1)
    dh = c_dim // NH
    inv = 1.0 / jnp.sqrt(jnp.float32(dh))
    outs = []
    for hh in range(NH):
        q = qkv[:, hh * dh:(hh + 1) * dh]
        k = qkv[:, c_dim + hh * dh:c_dim + (hh + 1) * dh]
        v = qkv[:, 2 * c_dim + hh * dh:2 * c_dim + (hh + 1) * dh]
        att = lax.dot_general(q, k, (((1,), (1,)), ((), ())),
                              precision=_PREC_MAIN,
                              preferred_element_type=jnp.float32) * inv
        att = att - jnp.max(att, axis=1, keepdims=True)
        att = jnp.exp(att)
        att = att / jnp.sum(att, axis=1, keepdims=True)
        outs.append(_dotd(att, v))
    o = jnp.concatenate(outs, axis=1)
    o = _dotd(o, aow_ref[...]) + aob_ref[...].reshape(1, c_dim) + zs
    mu = jnp.mean(o, axis=1, keepdims=True)
    var = jnp.mean((o - mu) ** 2, axis=1, keepdims=True)
    zo = (o - mu) * lax.rsqrt(var + 1e-5)
    zo = zo * lnw_ref[...].reshape(1, c_dim) + lnb_ref[...].reshape(1, c_dim)
    dense = lax.dot_general(zo, oh_ref[...], (((0,), (0,)), ((), ())),
                            precision=_PREC_MAIN,
                            preferred_element_type=jnp.float32)
    out_ref[...] = (x2d_ref[...].reshape(c_dim, n_tok) + dense).reshape(
        1, c_dim, n_tok)


def kernel(x, params, current_epoch):
    b, t_len, c_dim, h, w = x.shape
    n_tok = h * w
    k_sel = max(1, n_tok // 2)
    p = params

    x2d = x.reshape(t_len * c_dim, n_tok)
    x_nm = x2d.T
    w1 = p['r_w1'].transpose(2, 3, 0, 1).reshape(9 * p['r_w1'].shape[0],
                                                 c_dim)

    scr, scc = pl.pallas_call(
        functools.partial(_scores_kernel, n_tok=n_tok, h=h, w=w,
                          t_len=t_len, c_dim=c_dim),
        out_shape=(jax.ShapeDtypeStruct((1, n_tok), jnp.float32),
                   jax.ShapeDtypeStruct((n_tok, 1), jnp.float32)),
    )(x2d, w1, p['r_b1'].reshape(-1, 1), p['r_w2'].reshape(-1, 1),
      p['r_b2'].reshape(1, 1))

    rb = 128
    mrow, mcol = pl.pallas_call(
        functools.partial(_mask_kernel, n_tok=n_tok, k_sel=k_sel, rb=rb),
        grid=(n_tok // rb,),
        in_specs=[
            pl.BlockSpec((1, n_tok), lambda i: (0, 0)),
            pl.BlockSpec((n_tok, 1), lambda i: (0, 0)),
        ],
        out_specs=(pl.BlockSpec((1, rb), lambda i: (0, i)),
                   pl.BlockSpec((rb, 1), lambda i: (i, 0))),
        out_shape=(jax.ShapeDtypeStruct((1, n_tok), jnp.float32),
                   jax.ShapeDtypeStruct((n_tok, 1), jnp.float32)),
    )(scr, scc)

    prow = pl.pallas_call(
        functools.partial(_pos_kernel, n_tok=n_tok, rb=rb),
        grid=(n_tok // rb,),
        in_specs=[pl.BlockSpec((n_tok, 1), lambda i: (0, 0))],
        out_specs=pl.BlockSpec((1, rb), lambda i: (0, i)),
        out_shape=jax.ShapeDtypeStruct((1, n_tok), jnp.float32),
    )(mcol)

    kb2 = 128
    oh, ohs = pl.pallas_call(
        functools.partial(_oh_kernel, n_tok=n_tok, kb=kb2),
        grid=(k_sel // kb2,),
        in_specs=[
            pl.BlockSpec((1, n_tok), lambda i: (0, 0)),
            pl.BlockSpec((1, n_tok), lambda i: (0, 0)),
            pl.BlockSpec((1, n_tok), lambda i: (0, 0)),
        ],
        out_specs=(pl.BlockSpec((kb2, n_tok), lambda i: (i, 0)),
                   pl.BlockSpec((kb2, n_tok), lambda i: (i, 0))),
        out_shape=(jax.ShapeDtypeStruct((k_sel, n_tok), jnp.float32),
                   jax.ShapeDtypeStruct((k_sel, n_tok), jnp.float32)),
    )(scr, mrow, prow)

    kb = 128
    nblk = k_sel // kb
    z = pl.pallas_call(
        functools.partial(_mamba_kernel, kb=kb, t_len=t_len, c_dim=c_dim),
        grid=(nblk,),
        in_specs=[
            pl.BlockSpec((kb, n_tok), lambda i: (i, 0)),
            pl.BlockSpec((n_tok, t_len * c_dim), lambda i: (0, 0)),
            pl.BlockSpec((1, c_dim), lambda i: (0, 0)),
            pl.BlockSpec((c_dim, 2 * DI), lambda i: (0, 0)),
            pl.BlockSpec((DI, DC), lambda i: (0, 0)),
            pl.BlockSpec((1, DI), lambda i: (0, 0)),
            pl.BlockSpec((DI, DR + 2 * DS), lambda i: (0, 0)),
            pl.BlockSpec((DR, DI), lambda i: (0, 0)),
            pl.BlockSpec((1, DI), lambda i: (0, 0)),
            pl.BlockSpec((DS, DI), lambda i: (0, 0)),
            pl.BlockSpec((1, DI), lambda i: (0, 0)),
            pl.BlockSpec((DI, c_dim), lambda i: (0, 0)),
        ],
        out_specs=pl.BlockSpec((t_len, kb, c_dim), lambda i: (0, i, 0)),
        out_shape=jax.ShapeDtypeStruct((t_len, k_sel, c_dim), jnp.float32),
    )(ohs, x_nm, p['norm1_w'].reshape(1, c_dim), p['m_in_w'].T,
      p['m_conv_w'].reshape(DI, DC), p['m_conv_b'].reshape(1, DI),
      p['m_xproj_w'].T, p['m_dt_w'].T, p['m_dt_b'].reshape(1, DI),
      p['m_Alog'].T, p['m_D'].reshape(1, DI), p['m_out_w'].T)

    x3 = x2d.reshape(t_len, c_dim, n_tok)

    out = pl.pallas_call(
        functools.partial(_attn_kernel, k_sel=k_sel, c_dim=c_dim,
                          n_tok=n_tok),
        grid=(t_len,),
        in_specs=[
            pl.BlockSpec((1, k_sel, c_dim), lambda i: (i, 0, 0)),
            pl.BlockSpec((c_dim, 3 * c_dim), lambda i: (0, 0)),
            pl.BlockSpec((1, 3 * c_dim), lambda i: (0, 0)),
            pl.BlockSpec((c_dim, c_dim), lambda i: (0, 0)),
            pl.BlockSpec((1, c_dim), lambda i: (0, 0)),
            pl.BlockSpec((1, c_dim), lambda i: (0, 0)),
            pl.BlockSpec((1, c_dim), lambda i: (0, 0)),
            pl.BlockSpec((k_sel, n_tok), lambda i: (0, 0)),
            pl.BlockSpec((1, c_dim, n_tok), lambda i: (i, 0, 0)),
        ],
        out_specs=pl.BlockSpec((1, c_dim, n_tok), lambda i: (i, 0, 0)),
        out_shape=jax.ShapeDtypeStruct((t_len, c_dim, n_tok), jnp.float32),
    )(z, p['a_in_w'].T, p['a_in_b'].reshape(1, -1), p['a_out_w'].T,
      p['a_out_b'].reshape(1, c_dim), p['ln_w'].reshape(1, c_dim),
      p['ln_b'].reshape(1, c_dim), oh, x3)

    return out.reshape(b, t_len, c_dim, h, w)
